# baseline (speedup 1.0000x reference)
import jax
import jax.numpy as jnp
from jax import lax
from jax.experimental import pallas as pl
from jax.experimental.pallas import tpu as pltpu

T = 16384
K = 64
D_IN = 512
D_MODEL = 256
NUM_TYPES = 32
TYPE_DIM = 32
CONF_TH = 0.3

TB = 2048
NB = T // (2 * TB)
OFF = T // 2


def _body(t1_ref, t2_ref, durlat_ref, etype_ref, scal_ref, feat_ref,
          featb_ref, emb_ref, we_ref, ws_ref, wg_ref, b_ref,
          tok_ref, conf_ref, sk_ref,
          facc, sacc, emax):
    i = pl.program_id(0)

    @pl.when(i == 0)
    def _init():
        facc[...] = jnp.zeros_like(facc)
        sacc[...] = jnp.zeros_like(sacc)
        emax[...] = jnp.full_like(emax, -jnp.inf)

    t1 = t1_ref[...]
    t2 = t2_ref[...]

    def _half(base, fref):
        tt = base + lax.broadcasted_iota(jnp.int32, (K, TB), 1)
        mk = (tt >= t1) & (tt < t2)
        mf = mk.astype(jnp.float32)
        facc[...] += lax.dot_general(mf, fref[...],
                                     (((1,), (0,)), ((), ())),
                                     preferred_element_type=jnp.float32)
        eblk = scal_ref[0:1, pl.ds(base, TB)]
        elog = eblk * jnp.log(eblk + 1e-30)
        op2 = jnp.concatenate([scal_ref[0:5, pl.ds(base, TB)], elog,
                               jnp.zeros((2, TB), jnp.float32)], axis=0)
        sacc[...] += lax.dot_general(mf, op2,
                                     (((1,), (1,)), ((), ())),
                                     preferred_element_type=jnp.float32)
        cand = jnp.where(mk, eblk, -jnp.inf)
        emax[...] = jnp.maximum(emax[...], jnp.max(cand, axis=1, keepdims=True))
    _half(i * TB, feat_ref)
    _half(OFF + i * TB, featb_ref)

    @pl.when(i == NB - 1)
    def _finalize():
        cnt = (t2 - t1).astype(jnp.float32)
        sum_e = sacc[:, 0:1]
        s_safe = sum_e + 1e-6
        ent = (sum_e * jnp.log(s_safe) - sacc[:, 5:6]) / s_safe

        mean_motion = sum_e / cnt
        token_conf = sacc[:, 1:2] / cnt
        vis = sacc[:, 2:5] / cnt
        sk = jnp.log1p(jnp.concatenate(
            [durlat_ref[...], mean_motion, emax[...], ent, vis], axis=1))

        e_k = facc[...] / cnt
        onehot = (etype_ref[...] ==
                  lax.broadcasted_iota(jnp.int32, (K, NUM_TYPES), 1)
                  ).astype(jnp.float32)
        g = lax.dot_general(onehot, emb_ref[...], (((1,), (0,)), ((), ())),
                            preferred_element_type=jnp.float32)

        tokens = (lax.dot_general(e_k, we_ref[...], (((1,), (1,)), ((), ())),
                                  preferred_element_type=jnp.float32)
                  + lax.dot_general(sk, ws_ref[...], (((1,), (1,)), ((), ())),
                                    preferred_element_type=jnp.float32)
                  + lax.dot_general(g, wg_ref[...], (((1,), (1,)), ((), ())),
                                    preferred_element_type=jnp.float32)
                  + b_ref[...])
        tok_ref[...] = tokens
        conf_ref[...] = token_conf
        sk_ref[...] = sk


def kernel(features, energy, segments, frame_conf, stream_conf, fps,
           event_type_id, emb_table, W, b):
    t1 = segments[:, 0]
    t2 = segments[:, 1]
    fpsf = jnp.asarray(fps, jnp.float32)
    durlat = jnp.stack([(t2 - t1).astype(jnp.float32) / fpsf,
                        t1.astype(jnp.float32) / fpsf], axis=1)
    scal = jnp.concatenate([energy[None], frame_conf[None],
                            stream_conf.T], axis=0)
    whole = lambda shp: pl.BlockSpec(shp, lambda i: (0, 0))
    out = pl.pallas_call(
        _body,
        grid=(NB,),
        in_specs=[
            whole((K, 1)),
            whole((K, 1)),
            whole((K, 2)),
            whole((K, 1)),
            whole((5, T)),
            pl.BlockSpec((TB, D_IN), lambda i: (i, 0)),
            pl.BlockSpec((TB, D_IN), lambda i: (i + NB, 0)),
            whole((NUM_TYPES, TYPE_DIM)),
            whole((D_MODEL, D_IN)),
            whole((D_MODEL, 8)),
            whole((D_MODEL, TYPE_DIM)),
            whole((1, D_MODEL)),
        ],
        out_specs=[
            whole((K, D_MODEL)),
            whole((K, 1)),
            whole((K, 8)),
        ],
        out_shape=[
            jax.ShapeDtypeStruct((K, D_MODEL), jnp.float32),
            jax.ShapeDtypeStruct((K, 1), jnp.float32),
            jax.ShapeDtypeStruct((K, 8), jnp.float32),
        ],
        scratch_shapes=[
            pltpu.VMEM((K, D_IN), jnp.float32),
            pltpu.VMEM((K, 8), jnp.float32),
            pltpu.VMEM((K, 1), jnp.float32),
        ],
    )(t1[:, None], t2[:, None], durlat, event_type_id[:, None], scal,
      features, features, emb_table, W[:, :D_IN], W[:, D_IN:D_IN + 8],
      W[:, D_IN + 8:], b[None, :])

    tokens, conf2, sk = out
    token_conf = conf2[:, 0]
    attn_mask = token_conf >= CONF_TH
    return tokens, attn_mask, event_type_id, token_conf, sk


# speedup vs baseline: 1.0585x; 1.0585x over previous
import jax
import jax.numpy as jnp
from jax import lax
from jax.experimental import pallas as pl
from jax.experimental.pallas import tpu as pltpu

T = 16384
K = 64
D_IN = 512
D_MODEL = 256
NUM_TYPES = 32
TYPE_DIM = 32
CONF_TH = 0.3

TB = 4096
NB = T // TB


def _body(t1_ref, t2_ref, durlat_ref, etype_ref, scal_ref, feat_ref,
          emb_ref, we_ref, ws_ref, wg_ref, b_ref,
          tok_ref, conf_ref, sk_ref,
          facc, sacc, emax):
    i = pl.program_id(0)

    @pl.when(i == 0)
    def _init():
        facc[...] = jnp.zeros_like(facc)
        sacc[...] = jnp.zeros_like(sacc)
        emax[...] = jnp.full_like(emax, -jnp.inf)

    t1 = t1_ref[...]
    t2 = t2_ref[...]

    def _accum():
        base = i * TB
        tt = base + lax.broadcasted_iota(jnp.int32, (K, TB), 1)
        mk = (tt >= t1) & (tt < t2)
        mf = mk.astype(jnp.float32)
        facc[...] += lax.dot_general(mf, feat_ref[...],
                                     (((1,), (0,)), ((), ())),
                                     preferred_element_type=jnp.float32)
        eblk = scal_ref[0:1, pl.ds(base, TB)]
        elog = eblk * jnp.log(eblk + 1e-30)
        op2 = jnp.concatenate([scal_ref[0:5, pl.ds(base, TB)], elog,
                               jnp.zeros((2, TB), jnp.float32)], axis=0)
        sacc[...] += lax.dot_general(mf, op2,
                                     (((1,), (1,)), ((), ())),
                                     preferred_element_type=jnp.float32)
        cand = jnp.where(mk, eblk, -jnp.inf)
        emax[...] = jnp.maximum(emax[...], jnp.max(cand, axis=1, keepdims=True))
    _accum()

    @pl.when(i == NB - 1)
    def _finalize():
        cnt = (t2 - t1).astype(jnp.float32)
        sum_e = sacc[:, 0:1]
        s_safe = sum_e + 1e-6
        ent = (sum_e * jnp.log(s_safe) - sacc[:, 5:6]) / s_safe

        mean_motion = sum_e / cnt
        token_conf = sacc[:, 1:2] / cnt
        vis = sacc[:, 2:5] / cnt
        sk = jnp.log1p(jnp.concatenate(
            [durlat_ref[...], mean_motion, emax[...], ent, vis], axis=1))

        e_k = facc[...] / cnt
        onehot = (etype_ref[...] ==
                  lax.broadcasted_iota(jnp.int32, (K, NUM_TYPES), 1)
                  ).astype(jnp.float32)
        g = lax.dot_general(onehot, emb_ref[...], (((1,), (0,)), ((), ())),
                            preferred_element_type=jnp.float32)

        tokens = (lax.dot_general(e_k, we_ref[...], (((1,), (1,)), ((), ())),
                                  preferred_element_type=jnp.float32)
                  + lax.dot_general(sk, ws_ref[...], (((1,), (1,)), ((), ())),
                                    preferred_element_type=jnp.float32)
                  + lax.dot_general(g, wg_ref[...], (((1,), (1,)), ((), ())),
                                    preferred_element_type=jnp.float32)
                  + b_ref[...])
        tok_ref[...] = tokens
        conf_ref[...] = token_conf
        sk_ref[...] = sk


def kernel(features, energy, segments, frame_conf, stream_conf, fps,
           event_type_id, emb_table, W, b):
    t1 = segments[:, 0]
    t2 = segments[:, 1]
    fpsf = jnp.asarray(fps, jnp.float32)
    durlat = jnp.stack([(t2 - t1).astype(jnp.float32) / fpsf,
                        t1.astype(jnp.float32) / fpsf], axis=1)
    scal = jnp.concatenate([energy[None], frame_conf[None],
                            stream_conf.T], axis=0)
    whole = lambda shp: pl.BlockSpec(shp, lambda i: (0, 0))
    out = pl.pallas_call(
        _body,
        grid=(NB,),
        in_specs=[
            whole((K, 1)),
            whole((K, 1)),
            whole((K, 2)),
            whole((K, 1)),
            whole((5, T)),
            pl.BlockSpec((TB, D_IN), lambda i: (i, 0)),
            whole((NUM_TYPES, TYPE_DIM)),
            whole((D_MODEL, D_IN)),
            whole((D_MODEL, 8)),
            whole((D_MODEL, TYPE_DIM)),
            whole((1, D_MODEL)),
        ],
        out_specs=[
            whole((K, D_MODEL)),
            whole((K, 1)),
            whole((K, 8)),
        ],
        out_shape=[
            jax.ShapeDtypeStruct((K, D_MODEL), jnp.float32),
            jax.ShapeDtypeStruct((K, 1), jnp.float32),
            jax.ShapeDtypeStruct((K, 8), jnp.float32),
        ],
        scratch_shapes=[
            pltpu.VMEM((K, D_IN), jnp.float32),
            pltpu.VMEM((K, 8), jnp.float32),
            pltpu.VMEM((K, 1), jnp.float32),
        ],
    )(t1[:, None], t2[:, None], durlat, event_type_id[:, None], scal,
      features, emb_table, W[:, :D_IN], W[:, D_IN:D_IN + 8],
      W[:, D_IN + 8:], b[None, :])

    tokens, conf2, sk = out
    token_conf = conf2[:, 0]
    attn_mask = token_conf >= CONF_TH
    return tokens, attn_mask, event_type_id, token_conf, sk
